# baseline (speedup 1.0000x reference)
import functools

import jax
import jax.numpy as jnp
from jax import lax
from jax.experimental import pallas as pl
from jax.experimental.pallas import tpu as pltpu
from jax.experimental.pallas import tpu_sc as plsc

N = 10000
E = 320000
F = 128
H = 128
G = 64
EPS = 1e-5

NP_ = 10240
NC = 2
NS = 16
EPT = E // NS
K = 128
CH = 160
CHB = 16
NB = CH // CHB
UN = 8
RPS = NP_ // NS
DEGW = 16

_relu = lambda x: jnp.maximum(x, 0.0)


def _dot(a, b):
    return jnp.dot(a, b, preferred_element_type=jnp.float32)


def _deg_body(dst_hbm, ones_hbm, zeros_hbm, out_hbm, dst_v, ones_v, dacc):
    c = lax.axis_index("c")
    s = lax.axis_index("s")
    pltpu.sync_copy(dst_hbm.at[c, s], dst_v)
    pltpu.sync_copy(ones_hbm, ones_v)
    r0 = s * RPS
    pltpu.sync_copy(zeros_hbm.at[pl.ds(r0, RPS)], dacc.at[pl.ds(r0, RPS)])
    plsc.subcore_barrier()

    def body(j, carry):
        pltpu.sync_copy(ones_v, dacc.at[dst_v.at[j]], add=True)
        return carry

    lax.fori_loop(0, CH, body, 0)
    plsc.subcore_barrier()
    pltpu.sync_copy(dacc.at[pl.ds(r0, RPS)], out_hbm.at[c, pl.ds(r0, RPS)])


def _msg_body(y_hbm, src_hbm, dst_hbm, out_hbm,
              src_v, dst_v, rows, gsems, ssems, acc):
    c = lax.axis_index("c")
    s = lax.axis_index("s")
    r0 = s * RPS
    pltpu.sync_copy(y_hbm.at[pl.ds(c * NP_ + r0, RPS)], acc.at[pl.ds(r0, RPS)])
    plsc.subcore_barrier()

    def blk(blkid, carry):
        pltpu.sync_copy(src_hbm.at[c, s, pl.ds(blkid * CHB, CHB)], src_v)
        pltpu.sync_copy(dst_hbm.at[c, s, pl.ds(blkid * CHB, CHB)], dst_v)

        def grp(q, carry2):
            j0 = q * UN
            g = [None] * UN
            g[0] = pltpu.async_copy(y_hbm.at[src_v.at[j0]], rows[0],
                                    gsems[0])
            g[1] = pltpu.async_copy(y_hbm.at[src_v.at[j0 + 1]], rows[1],
                                    gsems[1])
            for jj in range(UN):
                b = jj & 1
                g[jj].wait()
                pltpu.sync_copy(rows[b], acc.at[dst_v.at[j0 + jj]], add=True)
                if jj + 2 < UN:
                    g[jj + 2] = pltpu.async_copy(
                        y_hbm.at[src_v.at[j0 + jj + 2]], rows[b], gsems[b])
            return carry2

        lax.fori_loop(0, CHB // UN, grp, 0)
        return carry

    lax.fori_loop(0, NB, blk, 0)
    plsc.subcore_barrier()
    pltpu.sync_copy(acc.at[pl.ds(r0, RPS)], out_hbm.at[c, pl.ds(r0, RPS)])


@functools.lru_cache(maxsize=None)
def _sc_calls():
    mesh = plsc.VectorSubcoreMesh(core_axis_name="c", subcore_axis_name="s")
    deg_call = pl.kernel(
        _deg_body,
        mesh=mesh,
        out_type=jax.ShapeDtypeStruct((NC, NP_, DEGW), jnp.float32),
        scratch_types=[
            pltpu.VMEM((CH, K), jnp.int32),
            pltpu.VMEM((K, DEGW), jnp.float32),
            pltpu.VMEM_SHARED((NP_, DEGW), jnp.float32),
        ],
    )
    msg_call = pl.kernel(
        _msg_body,
        mesh=mesh,
        out_type=jax.ShapeDtypeStruct((NC, NP_, H), jnp.float32),
        scratch_types=[
            pltpu.VMEM((CHB, K), jnp.int32),
            pltpu.VMEM((CHB, K), jnp.int32),
            [pltpu.VMEM((K, H), jnp.float32)] * 2,
            [pltpu.SemaphoreType.DMA] * 2,
            [pltpu.SemaphoreType.DMA] * 2,
            pltpu.VMEM_SHARED((NP_, H), jnp.float32),
        ],
    )
    return deg_call, msg_call


def _dinv_of(deg_blk):
    return lax.rsqrt(deg_blk[:N, 0:1] + 1.0)


def _t1_body(x_ref, deg_ref, g_ref, bb_ref, W1_ref, b1_ref, W2_ref, b2_ref,
             Wg_ref, y_ref):
    x = x_ref[0]
    m = jnp.mean(x, axis=0, keepdims=True)
    xc = x - m
    v = jnp.mean(xc * xc, axis=0, keepdims=True)
    xn = xc * lax.rsqrt(v + EPS) * g_ref[...] + bb_ref[...]
    h = _relu(_dot(xn, W1_ref[0]) + b1_ref[0])
    h = _relu(_dot(h, W2_ref[0]) + b2_ref[0])
    dinv = _dinv_of(deg_ref[0])
    y_ref[:N] = _dot(h * dinv, Wg_ref[0])
    y_ref[N:] = jnp.zeros((NP_ - N, H), jnp.float32)


def _t2_body(acc_ref, deg_ref, bg_ref, Wg_ref, y_ref):
    dinv = _dinv_of(deg_ref[0])
    h = _relu(acc_ref[0][:N] * dinv + bg_ref[0])
    y_ref[:N] = _dot(h * dinv, Wg_ref[0])
    y_ref[N:] = jnp.zeros((NP_ - N, H), jnp.float32)


def _t4_body(acc_ref, deg_ref, bg_ref, batch_ref, W1_ref, b1_ref, W2_ref,
             b2_ref, W3_ref, b3_ref, out_ref):
    pooled = []
    ids = lax.broadcasted_iota(jnp.int32, (G, N), 0)
    for b in range(2):
        dinv = _dinv_of(deg_ref[b])
        h = _relu(acc_ref[b, :N] * dinv + bg_ref[b])
        P = (batch_ref[b] == ids).astype(jnp.float32)
        cnt = jnp.sum(P, axis=1, keepdims=True)
        pooled.append(_dot(P, h) / jnp.maximum(cnt, 1.0))
    h = jnp.concatenate(pooled, axis=1)
    h = _relu(_dot(h, W1_ref[...]) + b1_ref[...])
    h = _relu(_dot(h, W2_ref[...]) + b2_ref[...])
    out_ref[...] = _dot(h, W3_ref[...]) + b3_ref[...]


_b0 = lambda b: (b, 0, 0)
_s0 = lambda b: (0, 0)

_TC_DEFS = {
    "t1": dict(
        body=_t1_body,
        grid=(2,),
        in_specs=[
            pl.BlockSpec((1, N, F), _b0),
            pl.BlockSpec((1, NP_, DEGW), _b0),
            pl.BlockSpec((1, F), _s0),
            pl.BlockSpec((1, F), _s0),
            pl.BlockSpec((1, F, 2 * H), _b0),
            pl.BlockSpec((1, 1, 2 * H), _b0),
            pl.BlockSpec((1, 2 * H, H), _b0),
            pl.BlockSpec((1, 1, H), _b0),
            pl.BlockSpec((1, H, H), _b0),
        ],
        out_specs=pl.BlockSpec((NP_, H), lambda b: (b, 0)),
        out_shape=jax.ShapeDtypeStruct((NC * NP_, H), jnp.float32),
    ),
    "t2": dict(
        body=_t2_body,
        grid=(2,),
        in_specs=[
            pl.BlockSpec((1, NP_, H), _b0),
            pl.BlockSpec((1, NP_, DEGW), _b0),
            pl.BlockSpec((1, 1, H), _b0),
            pl.BlockSpec((1, H, H), _b0),
        ],
        out_specs=pl.BlockSpec((NP_, H), lambda b: (b, 0)),
        out_shape=jax.ShapeDtypeStruct((NC * NP_, H), jnp.float32),
    ),
    "t4": dict(
        body=_t4_body,
        grid=None,
        in_specs=None,
        out_specs=None,
        out_shape=jax.ShapeDtypeStruct((G, 1), jnp.float32),
    ),
}


def _make_tc(name, interpret=False):
    d = _TC_DEFS[name]
    kw = {}
    if d["grid"] is not None:
        kw = dict(grid=d["grid"], in_specs=d["in_specs"], out_specs=d["out_specs"])
    return pl.pallas_call(d["body"], out_shape=d["out_shape"], interpret=interpret, **kw)


_t1_call = _make_tc("t1")
_t2_call = _make_tc("t2")
_t4_call = _make_tc("t4")


def _edge_blocks(ei, branch):
    src = ei[0].astype(jnp.int32) + branch * NP_
    dst = ei[1].astype(jnp.int32)
    pad = CH * K - EPT
    src = src.reshape(NS, EPT)
    dst = dst.reshape(NS, EPT)
    src = jnp.concatenate(
        [src, jnp.full((NS, pad), branch * NP_ + N, jnp.int32)], axis=1)
    dst = jnp.concatenate([dst, jnp.full((NS, pad), N, jnp.int32)], axis=1)
    return src.reshape(NS, CH, K), dst.reshape(NS, CH, K)


def kernel(xd_x, xp_x, xd_edge_index, xp_edge_index, xd_batch, xp_batch,
           bn_g, bn_b, Wd1, bd1, Wd2, bd2, Wgd1, bgd1, Wgd2, bgd2, Wgd3, bgd3,
           Wp1, bp1, Wp2, bp2, Wgp1, bgp1, Wgp2, bgp2, Wgp3, bgp3,
           W1, b1, W2, b2, W3, b3):
    sd, dd = _edge_blocks(xd_edge_index, 0)
    sp, dp = _edge_blocks(xp_edge_index, 1)
    src_idx = jnp.stack([sd, sp])
    dst_idx = jnp.stack([dd, dp])

    deg_call, msg_call = _sc_calls()
    ones = jnp.ones((K, DEGW), jnp.float32)
    zeros = jnp.zeros((NP_, DEGW), jnp.float32)
    deg = deg_call(dst_idx, ones, zeros)

    xs = jnp.stack([xd_x, xp_x])
    W1s = jnp.stack([Wd1, Wp1])
    b1s = jnp.stack([bd1, bp1]).reshape(NC, 1, 2 * H)
    W2s = jnp.stack([Wd2, Wp2])
    b2s = jnp.stack([bd2, bp2]).reshape(NC, 1, H)
    Wg1s = jnp.stack([Wgd1, Wgp1])
    Wg2s = jnp.stack([Wgd2, Wgp2])
    Wg3s = jnp.stack([Wgd3, Wgp3])
    bg1s = jnp.stack([bgd1, bgp1]).reshape(NC, 1, H)
    bg2s = jnp.stack([bgd2, bgp2]).reshape(NC, 1, H)
    bg3s = jnp.stack([bgd3, bgp3]).reshape(NC, 1, H)
    batch = jnp.stack([xd_batch, xp_batch]).reshape(NC, 1, N)

    y = _t1_call(xs, deg, bn_g.reshape(1, F), bn_b.reshape(1, F),
                 W1s, b1s, W2s, b2s, Wg1s)
    acc = msg_call(y, src_idx, dst_idx)
    y = _t2_call(acc, deg, bg1s, Wg2s)
    acc = msg_call(y, src_idx, dst_idx)
    y = _t2_call(acc, deg, bg2s, Wg3s)
    acc = msg_call(y, src_idx, dst_idx)
    return _t4_call(acc, deg, bg3s, batch,
                    W1, b1.reshape(1, 256), W2, b2.reshape(1, 128),
                    W3, b3.reshape(1, 1))


# speedup vs baseline: 1.6246x; 1.6246x over previous
import functools

import jax
import jax.numpy as jnp
from jax import lax
from jax.experimental import pallas as pl
from jax.experimental.pallas import tpu as pltpu
from jax.experimental.pallas import tpu_sc as plsc

N = 10000
E = 320000
F = 128
H = 128
G = 64
EPS = 1e-5

NP_ = 10240
NC = 2
NS = 16
EPT = E // NS
K = 128
CH = 160
CHB = 16
NB = CH // CHB
UN = 8
RPS = NP_ // NS
DEGW = 16

_relu = lambda x: jnp.maximum(x, 0.0)


def _dot(a, b):
    return jnp.dot(a, b, preferred_element_type=jnp.float32)


def _deg_body(dst_hbm, ones_hbm, zeros_hbm, out_hbm, dst_v, ones_v, dacc):
    c = lax.axis_index("c")
    s = lax.axis_index("s")
    pltpu.sync_copy(dst_hbm.at[c, s], dst_v)
    pltpu.sync_copy(ones_hbm, ones_v)
    r0 = s * RPS
    pltpu.sync_copy(zeros_hbm.at[pl.ds(r0, RPS)], dacc.at[pl.ds(r0, RPS)])
    plsc.subcore_barrier()

    def body(j, carry):
        pltpu.sync_copy(ones_v, dacc.at[dst_v.at[j]], add=True)
        return carry

    lax.fori_loop(0, CH, body, 0)
    plsc.subcore_barrier()
    pltpu.sync_copy(dacc.at[pl.ds(r0, RPS)], out_hbm.at[c, pl.ds(r0, RPS)])


def _msg_body(y_hbm, src_hbm, dst_hbm, out_hbm,
              src_v, dst_v, rows, gsems, ssems, acc):
    c = lax.axis_index("c")
    s = lax.axis_index("s")
    r0 = s * RPS
    pltpu.sync_copy(y_hbm.at[pl.ds(c * NP_ + r0, RPS)], acc.at[pl.ds(r0, RPS)])
    plsc.subcore_barrier()

    def blk(blkid, carry):
        pltpu.sync_copy(src_hbm.at[c, s, pl.ds(blkid * CHB, CHB)], src_v)
        pltpu.sync_copy(dst_hbm.at[c, s, pl.ds(blkid * CHB, CHB)], dst_v)

        def pair(q, carry2):
            j0 = 2 * q
            cp0 = pltpu.async_copy(y_hbm.at[src_v.at[j0]], rows[0], gsems[0])
            cp1 = pltpu.async_copy(y_hbm.at[src_v.at[j0 + 1]], rows[1],
                                   gsems[1])
            cp0.wait()
            pltpu.sync_copy(rows[0], acc.at[dst_v.at[j0]], add=True)
            cp1.wait()
            pltpu.sync_copy(rows[1], acc.at[dst_v.at[j0 + 1]], add=True)
            return carry2

        lax.fori_loop(0, CHB // 2, pair, 0)
        return carry

    lax.fori_loop(0, NB, blk, 0)
    plsc.subcore_barrier()
    pltpu.sync_copy(acc.at[pl.ds(r0, RPS)], out_hbm.at[c, pl.ds(r0, RPS)])


@functools.lru_cache(maxsize=None)
def _sc_calls():
    mesh = plsc.VectorSubcoreMesh(core_axis_name="c", subcore_axis_name="s")
    deg_call = pl.kernel(
        _deg_body,
        mesh=mesh,
        out_type=jax.ShapeDtypeStruct((NC, NP_, DEGW), jnp.float32),
        scratch_types=[
            pltpu.VMEM((CH, K), jnp.int32),
            pltpu.VMEM((K, DEGW), jnp.float32),
            pltpu.VMEM_SHARED((NP_, DEGW), jnp.float32),
        ],
    )
    msg_call = pl.kernel(
        _msg_body,
        mesh=mesh,
        out_type=jax.ShapeDtypeStruct((NC, NP_, H), jnp.float32),
        scratch_types=[
            pltpu.VMEM((CHB, K), jnp.int32),
            pltpu.VMEM((CHB, K), jnp.int32),
            [pltpu.VMEM((K, H), jnp.float32)] * 2,
            [pltpu.SemaphoreType.DMA] * 2,
            [pltpu.SemaphoreType.DMA] * 2,
            pltpu.VMEM_SHARED((NP_, H), jnp.float32),
        ],
    )
    return deg_call, msg_call


def _dinv_of(deg_blk):
    return lax.rsqrt(deg_blk[:N, 0:1] + 1.0)


def _t1_body(x_ref, deg_ref, g_ref, bb_ref, W1_ref, b1_ref, W2_ref, b2_ref,
             Wg_ref, y_ref):
    x = x_ref[0]
    m = jnp.mean(x, axis=0, keepdims=True)
    xc = x - m
    v = jnp.mean(xc * xc, axis=0, keepdims=True)
    xn = xc * lax.rsqrt(v + EPS) * g_ref[...] + bb_ref[...]
    h = _relu(_dot(xn, W1_ref[0]) + b1_ref[0])
    h = _relu(_dot(h, W2_ref[0]) + b2_ref[0])
    dinv = _dinv_of(deg_ref[0])
    y_ref[:N] = _dot(h * dinv, Wg_ref[0])
    y_ref[N:] = jnp.zeros((NP_ - N, H), jnp.float32)


def _t2_body(acc_ref, deg_ref, bg_ref, Wg_ref, y_ref):
    dinv = _dinv_of(deg_ref[0])
    h = _relu(acc_ref[0][:N] * dinv + bg_ref[0])
    y_ref[:N] = _dot(h * dinv, Wg_ref[0])
    y_ref[N:] = jnp.zeros((NP_ - N, H), jnp.float32)


def _t4_body(acc_ref, deg_ref, bg_ref, batch_ref, W1_ref, b1_ref, W2_ref,
             b2_ref, W3_ref, b3_ref, out_ref):
    pooled = []
    ids = lax.broadcasted_iota(jnp.int32, (G, N), 0)
    for b in range(2):
        dinv = _dinv_of(deg_ref[b])
        h = _relu(acc_ref[b, :N] * dinv + bg_ref[b])
        P = (batch_ref[b] == ids).astype(jnp.float32)
        cnt = jnp.sum(P, axis=1, keepdims=True)
        pooled.append(_dot(P, h) / jnp.maximum(cnt, 1.0))
    h = jnp.concatenate(pooled, axis=1)
    h = _relu(_dot(h, W1_ref[...]) + b1_ref[...])
    h = _relu(_dot(h, W2_ref[...]) + b2_ref[...])
    out_ref[...] = _dot(h, W3_ref[...]) + b3_ref[...]


_b0 = lambda b: (b, 0, 0)
_s0 = lambda b: (0, 0)

_TC_DEFS = {
    "t1": dict(
        body=_t1_body,
        grid=(2,),
        in_specs=[
            pl.BlockSpec((1, N, F), _b0),
            pl.BlockSpec((1, NP_, DEGW), _b0),
            pl.BlockSpec((1, F), _s0),
            pl.BlockSpec((1, F), _s0),
            pl.BlockSpec((1, F, 2 * H), _b0),
            pl.BlockSpec((1, 1, 2 * H), _b0),
            pl.BlockSpec((1, 2 * H, H), _b0),
            pl.BlockSpec((1, 1, H), _b0),
            pl.BlockSpec((1, H, H), _b0),
        ],
        out_specs=pl.BlockSpec((NP_, H), lambda b: (b, 0)),
        out_shape=jax.ShapeDtypeStruct((NC * NP_, H), jnp.float32),
    ),
    "t2": dict(
        body=_t2_body,
        grid=(2,),
        in_specs=[
            pl.BlockSpec((1, NP_, H), _b0),
            pl.BlockSpec((1, NP_, DEGW), _b0),
            pl.BlockSpec((1, 1, H), _b0),
            pl.BlockSpec((1, H, H), _b0),
        ],
        out_specs=pl.BlockSpec((NP_, H), lambda b: (b, 0)),
        out_shape=jax.ShapeDtypeStruct((NC * NP_, H), jnp.float32),
    ),
    "t4": dict(
        body=_t4_body,
        grid=None,
        in_specs=None,
        out_specs=None,
        out_shape=jax.ShapeDtypeStruct((G, 1), jnp.float32),
    ),
}


def _make_tc(name, interpret=False):
    d = _TC_DEFS[name]
    kw = {}
    if d["grid"] is not None:
        kw = dict(grid=d["grid"], in_specs=d["in_specs"], out_specs=d["out_specs"])
    return pl.pallas_call(d["body"], out_shape=d["out_shape"], interpret=interpret, **kw)


_t1_call = _make_tc("t1")
_t2_call = _make_tc("t2")
_t4_call = _make_tc("t4")


def _edge_blocks(ei, branch):
    src = ei[0].astype(jnp.int32) + branch * NP_
    dst = ei[1].astype(jnp.int32)
    pad = CH * K - EPT
    src = src.reshape(NS, EPT)
    dst = dst.reshape(NS, EPT)
    spread = N + (jnp.arange(pad, dtype=jnp.int32) % (NP_ - N))
    src = jnp.concatenate(
        [src, jnp.tile(branch * NP_ + spread, (NS, 1))], axis=1)
    dst = jnp.concatenate([dst, jnp.tile(spread, (NS, 1))], axis=1)
    return src.reshape(NS, CH, K), dst.reshape(NS, CH, K)


def kernel(xd_x, xp_x, xd_edge_index, xp_edge_index, xd_batch, xp_batch,
           bn_g, bn_b, Wd1, bd1, Wd2, bd2, Wgd1, bgd1, Wgd2, bgd2, Wgd3, bgd3,
           Wp1, bp1, Wp2, bp2, Wgp1, bgp1, Wgp2, bgp2, Wgp3, bgp3,
           W1, b1, W2, b2, W3, b3):
    sd, dd = _edge_blocks(xd_edge_index, 0)
    sp, dp = _edge_blocks(xp_edge_index, 1)
    src_idx = jnp.stack([sd, sp])
    dst_idx = jnp.stack([dd, dp])

    deg_call, msg_call = _sc_calls()
    ones = jnp.ones((K, DEGW), jnp.float32)
    zeros = jnp.zeros((NP_, DEGW), jnp.float32)
    deg = deg_call(dst_idx, ones, zeros)

    xs = jnp.stack([xd_x, xp_x])
    W1s = jnp.stack([Wd1, Wp1])
    b1s = jnp.stack([bd1, bp1]).reshape(NC, 1, 2 * H)
    W2s = jnp.stack([Wd2, Wp2])
    b2s = jnp.stack([bd2, bp2]).reshape(NC, 1, H)
    Wg1s = jnp.stack([Wgd1, Wgp1])
    Wg2s = jnp.stack([Wgd2, Wgp2])
    Wg3s = jnp.stack([Wgd3, Wgp3])
    bg1s = jnp.stack([bgd1, bgp1]).reshape(NC, 1, H)
    bg2s = jnp.stack([bgd2, bgp2]).reshape(NC, 1, H)
    bg3s = jnp.stack([bgd3, bgp3]).reshape(NC, 1, H)
    batch = jnp.stack([xd_batch, xp_batch]).reshape(NC, 1, N)

    y = _t1_call(xs, deg, bn_g.reshape(1, F), bn_b.reshape(1, F),
                 W1s, b1s, W2s, b2s, Wg1s)
    acc = msg_call(y, src_idx, dst_idx)
    y = _t2_call(acc, deg, bg1s, Wg2s)
    acc = msg_call(y, src_idx, dst_idx)
    y = _t2_call(acc, deg, bg2s, Wg3s)
    acc = msg_call(y, src_idx, dst_idx)
    return _t4_call(acc, deg, bg3s, batch,
                    W1, b1.reshape(1, 256), W2, b2.reshape(1, 128),
                    W3, b3.reshape(1, 1))


# speedup vs baseline: 1.6290x; 1.0027x over previous
import functools

import jax
import jax.numpy as jnp
from jax import lax
from jax.experimental import pallas as pl
from jax.experimental.pallas import tpu as pltpu
from jax.experimental.pallas import tpu_sc as plsc

N = 10000
E = 320000
F = 128
H = 128
G = 64
EPS = 1e-5

NP_ = 10240
NC = 2
NS = 16
EPT = E // NS
K = 128
CH = 160
CHB = 16
NB = CH // CHB
UN = 8
RPS = NP_ // NS
DEGW = 16

_relu = lambda x: jnp.maximum(x, 0.0)


def _dot(a, b):
    return jnp.dot(a, b, preferred_element_type=jnp.float32)


def _deg_body(dst_hbm, ones_hbm, zeros_hbm, out_hbm, dst_v, ones_v, dacc):
    c = lax.axis_index("c")
    s = lax.axis_index("s")
    pltpu.sync_copy(dst_hbm.at[c, s], dst_v)
    pltpu.sync_copy(ones_hbm, ones_v)
    r0 = s * RPS
    pltpu.sync_copy(zeros_hbm.at[pl.ds(r0, RPS)], dacc.at[pl.ds(r0, RPS)])
    plsc.subcore_barrier()

    def body(j, carry):
        pltpu.sync_copy(ones_v, dacc.at[dst_v.at[j]], add=True)
        return carry

    lax.fori_loop(0, CH, body, 0)
    plsc.subcore_barrier()
    pltpu.sync_copy(dacc.at[pl.ds(r0, RPS)], out_hbm.at[c, pl.ds(r0, RPS)])


def _msg_body(y_hbm, src_hbm, dst_hbm, out_hbm,
              src_v, dst_v, rows, gsems, ssems, acc):
    c = lax.axis_index("c")
    s = lax.axis_index("s")
    r0 = s * RPS
    pltpu.sync_copy(y_hbm.at[pl.ds(c * NP_ + r0, RPS)], acc.at[pl.ds(r0, RPS)])
    plsc.subcore_barrier()

    def blk(blkid, carry):
        pltpu.sync_copy(src_hbm.at[c, s, pl.ds(blkid * CHB, CHB)], src_v)
        pltpu.sync_copy(dst_hbm.at[c, s, pl.ds(blkid * CHB, CHB)], dst_v)

        def pair(q, carry2):
            j0 = 2 * q
            cp0 = pltpu.async_copy(y_hbm.at[src_v.at[j0]], rows[0], gsems[0])
            cp1 = pltpu.async_copy(y_hbm.at[src_v.at[j0 + 1]], rows[1],
                                   gsems[1])
            cp0.wait()
            pltpu.sync_copy(rows[0], acc.at[dst_v.at[j0]], add=True)
            cp1.wait()
            pltpu.sync_copy(rows[1], acc.at[dst_v.at[j0 + 1]], add=True)
            return carry2

        lax.fori_loop(0, CHB // 2, pair, 0)
        return carry

    lax.fori_loop(0, NB, blk, 0)
    plsc.subcore_barrier()
    pltpu.sync_copy(acc.at[pl.ds(r0, RPS)], out_hbm.at[c, pl.ds(r0, RPS)])


@functools.lru_cache(maxsize=None)
def _sc_calls():
    mesh = plsc.VectorSubcoreMesh(core_axis_name="c", subcore_axis_name="s")
    deg_call = pl.kernel(
        _deg_body,
        mesh=mesh,
        out_type=jax.ShapeDtypeStruct((NC, NP_, DEGW), jnp.float32),
        scratch_types=[
            pltpu.VMEM((CH, K), jnp.int32),
            pltpu.VMEM((K, DEGW), jnp.float32),
            pltpu.VMEM_SHARED((NP_, DEGW), jnp.float32),
        ],
    )
    msg_call = pl.kernel(
        _msg_body,
        mesh=mesh,
        out_type=jax.ShapeDtypeStruct((NC, NP_, H), jnp.float32),
        scratch_types=[
            pltpu.VMEM((CHB, K), jnp.int32),
            pltpu.VMEM((CHB, K), jnp.int32),
            [pltpu.VMEM((K, H), jnp.float32) for _ in range(2)],
            [pltpu.SemaphoreType.DMA for _ in range(2)],
            [pltpu.SemaphoreType.DMA for _ in range(2)],
            pltpu.VMEM_SHARED((NP_, H), jnp.float32),
        ],
    )
    return deg_call, msg_call


def _dinv_of(deg_blk):
    return lax.rsqrt(deg_blk[:N, 0:1] + 1.0)


def _t1_body(x_ref, deg_ref, g_ref, bb_ref, W1_ref, b1_ref, W2_ref, b2_ref,
             Wg_ref, y_ref):
    x = x_ref[0]
    m = jnp.mean(x, axis=0, keepdims=True)
    xc = x - m
    v = jnp.mean(xc * xc, axis=0, keepdims=True)
    xn = xc * lax.rsqrt(v + EPS) * g_ref[...] + bb_ref[...]
    h = _relu(_dot(xn, W1_ref[0]) + b1_ref[0])
    h = _relu(_dot(h, W2_ref[0]) + b2_ref[0])
    dinv = _dinv_of(deg_ref[0])
    y_ref[:N] = _dot(h * dinv, Wg_ref[0])
    y_ref[N:] = jnp.zeros((NP_ - N, H), jnp.float32)


def _t2_body(acc_ref, deg_ref, bg_ref, Wg_ref, y_ref):
    dinv = _dinv_of(deg_ref[0])
    h = _relu(acc_ref[0][:N] * dinv + bg_ref[0])
    y_ref[:N] = _dot(h * dinv, Wg_ref[0])
    y_ref[N:] = jnp.zeros((NP_ - N, H), jnp.float32)


def _t4_body(acc_ref, deg_ref, bg_ref, batch_ref, W1_ref, b1_ref, W2_ref,
             b2_ref, W3_ref, b3_ref, out_ref):
    pooled = []
    ids = lax.broadcasted_iota(jnp.int32, (G, N), 0)
    for b in range(2):
        dinv = _dinv_of(deg_ref[b])
        h = _relu(acc_ref[b, :N] * dinv + bg_ref[b])
        P = (batch_ref[b] == ids).astype(jnp.float32)
        cnt = jnp.sum(P, axis=1, keepdims=True)
        pooled.append(_dot(P, h) / jnp.maximum(cnt, 1.0))
    h = jnp.concatenate(pooled, axis=1)
    h = _relu(_dot(h, W1_ref[...]) + b1_ref[...])
    h = _relu(_dot(h, W2_ref[...]) + b2_ref[...])
    out_ref[...] = _dot(h, W3_ref[...]) + b3_ref[...]


_b0 = lambda b: (b, 0, 0)
_s0 = lambda b: (0, 0)

_TC_DEFS = {
    "t1": dict(
        body=_t1_body,
        grid=(2,),
        in_specs=[
            pl.BlockSpec((1, N, F), _b0),
            pl.BlockSpec((1, NP_, DEGW), _b0),
            pl.BlockSpec((1, F), _s0),
            pl.BlockSpec((1, F), _s0),
            pl.BlockSpec((1, F, 2 * H), _b0),
            pl.BlockSpec((1, 1, 2 * H), _b0),
            pl.BlockSpec((1, 2 * H, H), _b0),
            pl.BlockSpec((1, 1, H), _b0),
            pl.BlockSpec((1, H, H), _b0),
        ],
        out_specs=pl.BlockSpec((NP_, H), lambda b: (b, 0)),
        out_shape=jax.ShapeDtypeStruct((NC * NP_, H), jnp.float32),
    ),
    "t2": dict(
        body=_t2_body,
        grid=(2,),
        in_specs=[
            pl.BlockSpec((1, NP_, H), _b0),
            pl.BlockSpec((1, NP_, DEGW), _b0),
            pl.BlockSpec((1, 1, H), _b0),
            pl.BlockSpec((1, H, H), _b0),
        ],
        out_specs=pl.BlockSpec((NP_, H), lambda b: (b, 0)),
        out_shape=jax.ShapeDtypeStruct((NC * NP_, H), jnp.float32),
    ),
    "t4": dict(
        body=_t4_body,
        grid=None,
        in_specs=None,
        out_specs=None,
        out_shape=jax.ShapeDtypeStruct((G, 1), jnp.float32),
    ),
}


def _make_tc(name, interpret=False):
    d = _TC_DEFS[name]
    kw = {}
    if d["grid"] is not None:
        kw = dict(grid=d["grid"], in_specs=d["in_specs"], out_specs=d["out_specs"])
    return pl.pallas_call(d["body"], out_shape=d["out_shape"], interpret=interpret, **kw)


_t1_call = _make_tc("t1")
_t2_call = _make_tc("t2")
_t4_call = _make_tc("t4")


def _edge_blocks(ei, branch):
    src = ei[0].astype(jnp.int32) + branch * NP_
    dst = ei[1].astype(jnp.int32)
    pad = CH * K - EPT
    src = src.reshape(NS, EPT)
    dst = dst.reshape(NS, EPT)
    spread = N + (jnp.arange(pad, dtype=jnp.int32) % (NP_ - N))
    src = jnp.concatenate(
        [src, jnp.tile(branch * NP_ + spread, (NS, 1))], axis=1)
    dst = jnp.concatenate([dst, jnp.tile(spread, (NS, 1))], axis=1)
    return src.reshape(NS, CH, K), dst.reshape(NS, CH, K)


def kernel(xd_x, xp_x, xd_edge_index, xp_edge_index, xd_batch, xp_batch,
           bn_g, bn_b, Wd1, bd1, Wd2, bd2, Wgd1, bgd1, Wgd2, bgd2, Wgd3, bgd3,
           Wp1, bp1, Wp2, bp2, Wgp1, bgp1, Wgp2, bgp2, Wgp3, bgp3,
           W1, b1, W2, b2, W3, b3):
    sd, dd = _edge_blocks(xd_edge_index, 0)
    sp, dp = _edge_blocks(xp_edge_index, 1)
    src_idx = jnp.stack([sd, sp])
    dst_idx = jnp.stack([dd, dp])

    deg_call, msg_call = _sc_calls()
    ones = jnp.ones((K, DEGW), jnp.float32)
    zeros = jnp.zeros((NP_, DEGW), jnp.float32)
    deg = deg_call(dst_idx, ones, zeros)

    xs = jnp.stack([xd_x, xp_x])
    W1s = jnp.stack([Wd1, Wp1])
    b1s = jnp.stack([bd1, bp1]).reshape(NC, 1, 2 * H)
    W2s = jnp.stack([Wd2, Wp2])
    b2s = jnp.stack([bd2, bp2]).reshape(NC, 1, H)
    Wg1s = jnp.stack([Wgd1, Wgp1])
    Wg2s = jnp.stack([Wgd2, Wgp2])
    Wg3s = jnp.stack([Wgd3, Wgp3])
    bg1s = jnp.stack([bgd1, bgp1]).reshape(NC, 1, H)
    bg2s = jnp.stack([bgd2, bgp2]).reshape(NC, 1, H)
    bg3s = jnp.stack([bgd3, bgp3]).reshape(NC, 1, H)
    batch = jnp.stack([xd_batch, xp_batch]).reshape(NC, 1, N)

    y = _t1_call(xs, deg, bn_g.reshape(1, F), bn_b.reshape(1, F),
                 W1s, b1s, W2s, b2s, Wg1s)
    acc = msg_call(y, src_idx, dst_idx)
    y = _t2_call(acc, deg, bg1s, Wg2s)
    acc = msg_call(y, src_idx, dst_idx)
    y = _t2_call(acc, deg, bg2s, Wg3s)
    acc = msg_call(y, src_idx, dst_idx)
    return _t4_call(acc, deg, bg3s, batch,
                    W1, b1.reshape(1, 256), W2, b2.reshape(1, 128),
                    W3, b3.reshape(1, 1))
